# baseline (speedup 1.0000x reference)
import functools

import jax
import jax.numpy as jnp
from jax import lax
from jax.experimental import pallas as pl
from jax.experimental.pallas import tpu as pltpu
from jax.experimental.pallas import tpu_sc as plsc

VOCAB = 100000
EMBED = 32
BATCH = 1024

_NC = 2
_NS = 16
_NW = _NC * _NS
_B_PER_W = BATCH // _NW

_V_TILE = 2048


def _gather_body(table_hbm, idx_hbm, out_hbm, idx_v, rows_v, sem):
    wid = lax.axis_index("s") * _NC + lax.axis_index("c")
    base = wid * _B_PER_W
    pltpu.sync_copy(idx_hbm.at[pl.ds(base, _B_PER_W)], idx_v)
    pltpu.async_copy(table_hbm.at[idx_v], rows_v, sem).wait()
    pltpu.sync_copy(rows_v, out_hbm.at[pl.ds(base, _B_PER_W)])


@jax.jit
def _sc_gather(table, idx):
    mesh = plsc.VectorSubcoreMesh(core_axis_name="c", subcore_axis_name="s")
    return pl.kernel(
        _gather_body,
        mesh=mesh,
        out_type=jax.ShapeDtypeStruct((BATCH, EMBED), jnp.float32),
        scratch_types=[
            pltpu.VMEM((_B_PER_W,), jnp.int32),
            pltpu.VMEM((_B_PER_W, EMBED), jnp.float32),
            pltpu.SemaphoreType.DMA,
        ],
        compiler_params=pltpu.CompilerParams(use_tc_tiling_on_sc=False),
    )(table, idx)


def _proj_body(y_ref, w_ref, b_ref, o_ref):
    o_ref[...] = (
        lax.dot_general(
            y_ref[...],
            w_ref[...],
            (((1,), (1,)), ((), ())),
            preferred_element_type=jnp.float32,
        )
        + b_ref[...]
    )


@jax.jit
def _tc_project(y, W, b2):
    grid = pl.cdiv(VOCAB, _V_TILE)
    return pl.pallas_call(
        _proj_body,
        grid=(grid,),
        in_specs=[
            pl.BlockSpec((BATCH, EMBED), lambda j: (0, 0)),
            pl.BlockSpec((_V_TILE, EMBED), lambda j: (j, 0)),
            pl.BlockSpec((1, _V_TILE), lambda j: (0, j)),
        ],
        out_specs=pl.BlockSpec((BATCH, _V_TILE), lambda j: (0, j)),
        out_shape=jax.ShapeDtypeStruct((BATCH, VOCAB), jnp.float32),
    )(y, W, b2)


def kernel(batch, embed_table, W, b):
    y = _sc_gather(embed_table, batch.astype(jnp.int32))
    return _tc_project(y, W, b.reshape(1, VOCAB))


# speedup vs baseline: 2.6333x; 2.6333x over previous
import functools

import jax
import jax.numpy as jnp
from jax import lax
from jax.experimental import pallas as pl
from jax.experimental.pallas import tpu as pltpu
from jax.experimental.pallas import tpu_sc as plsc

VOCAB = 100000
EMBED = 32
BATCH = 1024

_NC = 2
_NS = 16
_NW = _NC * _NS

_V_TILE = 2048

_LANES = 16


def _gather_body(tableT_hbm, idx_hbm, outT_hbm, idx_v, row_v, out_v):
    wid = lax.axis_index("s") * _NC + lax.axis_index("c")
    pltpu.sync_copy(idx_hbm, idx_v)
    pltpu.sync_copy(tableT_hbm.at[wid], row_v)

    def body(i, carry):
        idx16 = idx_v[pl.ds(i * _LANES, _LANES)]
        out_v[pl.ds(i * _LANES, _LANES)] = plsc.load_gather(row_v, [idx16])
        return carry

    lax.fori_loop(0, BATCH // _LANES, body, 0)
    pltpu.sync_copy(out_v, outT_hbm.at[wid])


@jax.jit
def _sc_gather(tableT, idx):
    mesh = plsc.VectorSubcoreMesh(core_axis_name="c", subcore_axis_name="s")
    return pl.kernel(
        _gather_body,
        mesh=mesh,
        out_type=jax.ShapeDtypeStruct((EMBED, BATCH), jnp.float32),
        scratch_types=[
            pltpu.VMEM((BATCH,), jnp.int32),
            pltpu.VMEM((VOCAB,), jnp.float32),
            pltpu.VMEM((BATCH,), jnp.float32),
        ],
        compiler_params=pltpu.CompilerParams(
            use_tc_tiling_on_sc=False, needs_layout_passes=False
        ),
    )(tableT, idx)


def _proj_body(wt_ref, yt_ref, b_ref, o_ref):
    o_ref[...] = (
        lax.dot_general(
            wt_ref[...],
            yt_ref[...],
            (((0,), (0,)), ((), ())),
            preferred_element_type=jnp.float32,
        )
        + b_ref[...]
    )


@jax.jit
def _tc_project(wt, yt, b2):
    grid = pl.cdiv(VOCAB, _V_TILE)
    return pl.pallas_call(
        _proj_body,
        grid=(grid,),
        in_specs=[
            pl.BlockSpec((EMBED, _V_TILE), lambda j: (0, j)),
            pl.BlockSpec((EMBED, BATCH), lambda j: (0, 0)),
            pl.BlockSpec((_V_TILE, 1), lambda j: (j, 0)),
        ],
        out_specs=pl.BlockSpec((_V_TILE, BATCH), lambda j: (j, 0)),
        out_shape=jax.ShapeDtypeStruct((VOCAB, BATCH), jnp.float32),
    )(wt, yt, b2)


def kernel(batch, embed_table, W, b):
    yt = _sc_gather(embed_table.T, batch.astype(jnp.int32))
    outT = _tc_project(W.T, yt, b.reshape(VOCAB, 1))
    return outT.T


# speedup vs baseline: 2.9632x; 1.1253x over previous
import functools

import jax
import jax.numpy as jnp
from jax import lax
from jax.experimental import pallas as pl
from jax.experimental.pallas import tpu as pltpu
from jax.experimental.pallas import tpu_sc as plsc

VOCAB = 100000
EMBED = 32
BATCH = 1024

_NC = 2
_NS = 16
_NW = _NC * _NS
_B_PER_W = BATCH // _NW

_V_TILE = 2048


def _gather_body(table_hbm, idx_hbm, out_hbm, idx_v, rows_v, sem):
    wid = lax.axis_index("s") * _NC + lax.axis_index("c")
    base = wid * _B_PER_W
    pltpu.sync_copy(idx_hbm.at[pl.ds(base, _B_PER_W)], idx_v)
    pltpu.async_copy(table_hbm.at[idx_v], rows_v, sem).wait()
    pltpu.sync_copy(rows_v, out_hbm.at[pl.ds(base, _B_PER_W)])


@jax.jit
def _sc_gather(table, idx):
    mesh = plsc.VectorSubcoreMesh(core_axis_name="c", subcore_axis_name="s")
    return pl.kernel(
        _gather_body,
        mesh=mesh,
        out_type=jax.ShapeDtypeStruct((BATCH, EMBED), jnp.float32),
        scratch_types=[
            pltpu.VMEM((_B_PER_W,), jnp.int32),
            pltpu.VMEM((_B_PER_W, EMBED), jnp.float32),
            pltpu.SemaphoreType.DMA,
        ],
        compiler_params=pltpu.CompilerParams(use_tc_tiling_on_sc=False),
    )(table, idx)


def _proj_body(wt_ref, y_ref, b_ref, o_ref):
    ones = jnp.ones((1, BATCH), dtype=jnp.float32)
    o_ref[...] = (
        lax.dot_general(
            wt_ref[...],
            y_ref[...],
            (((0,), (1,)), ((), ())),
            preferred_element_type=jnp.float32,
        )
        + lax.dot_general(
            b_ref[...],
            ones,
            (((0,), (0,)), ((), ())),
            preferred_element_type=jnp.float32,
        )
    )


@jax.jit
def _tc_project(wt, y, b2):
    grid = pl.cdiv(VOCAB, _V_TILE)
    return pl.pallas_call(
        _proj_body,
        grid=(grid,),
        in_specs=[
            pl.BlockSpec((EMBED, _V_TILE), lambda j: (0, j)),
            pl.BlockSpec((BATCH, EMBED), lambda j: (0, 0)),
            pl.BlockSpec((1, _V_TILE), lambda j: (0, j)),
        ],
        out_specs=pl.BlockSpec((_V_TILE, BATCH), lambda j: (j, 0)),
        out_shape=jax.ShapeDtypeStruct((VOCAB, BATCH), jnp.float32),
    )(wt, y, b2)


def kernel(batch, embed_table, W, b):
    y = _sc_gather(embed_table, batch.astype(jnp.int32))
    outT = _tc_project(W.T, y, b.reshape(1, VOCAB))
    return outT.T


# speedup vs baseline: 3.4471x; 1.1633x over previous
import functools

import jax
import jax.numpy as jnp
from jax import lax
from jax.experimental import pallas as pl
from jax.experimental.pallas import tpu as pltpu
from jax.experimental.pallas import tpu_sc as plsc

VOCAB = 100000
EMBED = 32
BATCH = 1024

_NC = 2
_NS = 16
_NW = _NC * _NS

_CHUNK = 128
_NCHUNK = BATCH // _CHUNK
_LANES = 16

_V_TILE = 2048


def _gather_body(flat_hbm, idx_hbm, outT_hbm, out_v, sem, *idx_refs):
    wid = lax.axis_index("s") * _NC + lax.axis_index("c")
    off = wid * VOCAB
    for j in range(_NCHUNK):
        pltpu.sync_copy(idx_hbm.at[pl.ds(j * _CHUNK, _CHUNK)], idx_refs[j])
    for j in range(_NCHUNK):
        for i in range(_CHUNK // _LANES):
            sl = pl.ds(i * _LANES, _LANES)
            idx_refs[j][sl] = idx_refs[j][sl] + off
    copies = [
        pltpu.async_copy(
            flat_hbm.at[idx_refs[j]], out_v.at[pl.ds(j * _CHUNK, _CHUNK)], sem
        )
        for j in range(_NCHUNK)
    ]
    for c in copies:
        c.wait()
    pltpu.sync_copy(out_v, outT_hbm.at[wid])


@jax.jit
def _sc_gather(flat_table, idx):
    mesh = plsc.VectorSubcoreMesh(core_axis_name="c", subcore_axis_name="s")
    return pl.kernel(
        _gather_body,
        mesh=mesh,
        out_type=jax.ShapeDtypeStruct((EMBED, BATCH), jnp.float32),
        scratch_types=[
            pltpu.VMEM((BATCH,), jnp.float32),
            pltpu.SemaphoreType.DMA,
        ]
        + [pltpu.VMEM((_CHUNK,), jnp.int32) for _ in range(_NCHUNK)],
        compiler_params=pltpu.CompilerParams(use_tc_tiling_on_sc=False),
    )(flat_table, idx)


def _proj_body(wt_ref, yt_ref, b_ref, o_ref):
    ones = jnp.ones((1, BATCH), dtype=jnp.float32)
    o_ref[...] = (
        lax.dot_general(
            wt_ref[...],
            yt_ref[...],
            (((0,), (0,)), ((), ())),
            preferred_element_type=jnp.float32,
        )
        + lax.dot_general(
            b_ref[...],
            ones,
            (((0,), (0,)), ((), ())),
            preferred_element_type=jnp.float32,
        )
    )


@jax.jit
def _tc_project(wt, yt, b2):
    grid = pl.cdiv(VOCAB, _V_TILE)
    return pl.pallas_call(
        _proj_body,
        grid=(grid,),
        in_specs=[
            pl.BlockSpec((EMBED, _V_TILE), lambda j: (0, j)),
            pl.BlockSpec((EMBED, BATCH), lambda j: (0, 0)),
            pl.BlockSpec((1, _V_TILE), lambda j: (0, j)),
        ],
        out_specs=pl.BlockSpec((_V_TILE, BATCH), lambda j: (j, 0)),
        out_shape=jax.ShapeDtypeStruct((VOCAB, BATCH), jnp.float32),
    )(wt, yt, b2)


def kernel(batch, embed_table, W, b):
    flat = embed_table.T.reshape(-1)
    yt = _sc_gather(flat, batch.astype(jnp.int32))
    outT = _tc_project(W.T, yt, b.reshape(1, VOCAB))
    return outT.T


# speedup vs baseline: 3.4575x; 1.0030x over previous
import functools

import jax
import jax.numpy as jnp
from jax import lax
from jax.experimental import pallas as pl
from jax.experimental.pallas import tpu as pltpu
from jax.experimental.pallas import tpu_sc as plsc

VOCAB = 100000
EMBED = 32
BATCH = 1024

_NC = 2
_NS = 16
_NW = _NC * _NS

_CHUNK = 128
_NCHUNK = BATCH // _CHUNK
_LANES = 16

_V_TILE = 4096


def _gather_body(flat_hbm, idx_hbm, outT_hbm, out_v, sem, *idx_refs):
    wid = lax.axis_index("s") * _NC + lax.axis_index("c")
    off = wid * VOCAB
    for j in range(_NCHUNK):
        pltpu.sync_copy(idx_hbm.at[pl.ds(j * _CHUNK, _CHUNK)], idx_refs[j])
    for j in range(_NCHUNK):
        for i in range(_CHUNK // _LANES):
            sl = pl.ds(i * _LANES, _LANES)
            idx_refs[j][sl] = idx_refs[j][sl] + off
    copies = [
        pltpu.async_copy(
            flat_hbm.at[idx_refs[j]], out_v.at[pl.ds(j * _CHUNK, _CHUNK)], sem
        )
        for j in range(_NCHUNK)
    ]
    for c in copies:
        c.wait()
    pltpu.sync_copy(out_v, outT_hbm.at[wid])


@jax.jit
def _sc_gather(flat_table, idx):
    mesh = plsc.VectorSubcoreMesh(core_axis_name="c", subcore_axis_name="s")
    return pl.kernel(
        _gather_body,
        mesh=mesh,
        out_type=jax.ShapeDtypeStruct((EMBED, BATCH), jnp.float32),
        scratch_types=[
            pltpu.VMEM((BATCH,), jnp.float32),
            pltpu.SemaphoreType.DMA,
        ]
        + [pltpu.VMEM((_CHUNK,), jnp.int32) for _ in range(_NCHUNK)],
        compiler_params=pltpu.CompilerParams(use_tc_tiling_on_sc=False),
    )(flat_table, idx)


def _proj_body(wt_ref, yt_ref, b_ref, o_ref):
    ones = jnp.ones((1, BATCH), dtype=jnp.float32)
    o_ref[...] = (
        lax.dot_general(
            wt_ref[...],
            yt_ref[...],
            (((0,), (0,)), ((), ())),
            preferred_element_type=jnp.float32,
        )
        + lax.dot_general(
            b_ref[...],
            ones,
            (((0,), (0,)), ((), ())),
            preferred_element_type=jnp.float32,
        )
    )


@jax.jit
def _tc_project(wt, yt, b2):
    grid = pl.cdiv(VOCAB, _V_TILE)
    return pl.pallas_call(
        _proj_body,
        grid=(grid,),
        in_specs=[
            pl.BlockSpec((EMBED, _V_TILE), lambda j: (0, j)),
            pl.BlockSpec((EMBED, BATCH), lambda j: (0, 0)),
            pl.BlockSpec((1, _V_TILE), lambda j: (0, j)),
        ],
        out_specs=pl.BlockSpec((_V_TILE, BATCH), lambda j: (j, 0)),
        out_shape=jax.ShapeDtypeStruct((VOCAB, BATCH), jnp.float32),
    )(wt, yt, b2)


def kernel(batch, embed_table, W, b):
    flat = embed_table.T.reshape(-1)
    yt = _sc_gather(flat, batch.astype(jnp.int32))
    outT = _tc_project(W.T, yt, b.reshape(1, VOCAB))
    return outT.T
